# baseline (speedup 1.0000x reference)
import functools

import jax
import jax.numpy as jnp
from jax import lax
from jax.experimental import pallas as pl
from jax.experimental.pallas import tpu as pltpu
from jax.experimental.pallas import tpu_sc as plsc

_NC = 2
_NS = 16
_CH = 32


def _sc_body(vel_hbm, pe_hbm, out_hbm, vel_v, idx_lo, idx_hi, low_v, high_v,
             out_v, sem, *, max_len, seq_len):
    rows_per_worker = seq_len // (_NC * _NS)
    n_chunks = rows_per_worker // _CH
    wid = lax.axis_index("s") * _NC + lax.axis_index("c")
    base = wid * rows_per_worker

    pltpu.sync_copy(vel_hbm, vel_v)
    v = jnp.clip(vel_v[...], 0.0, 0.999)
    u = 1.0 - v * v
    r = jnp.full((16,), 1.0)
    for _ in range(14):
        r = r * (1.5 - 0.5 * u * r * r)
    inv_gamma = u * r
    top = jnp.float32(max_len - 1)

    def chunk_body(c, carry):
        t0 = pl.multiple_of(base + c * _CH, _CH)
        for j in range(_CH // 16):
            tv = (t0 + j * 16 + lax.iota(jnp.int32, 16)).astype(jnp.float32)
            relv = jnp.minimum(tv * inv_gamma, top)
            lowv = relv.astype(jnp.int32)
            idx_lo[pl.ds(j * 16, 16)] = lowv
            idx_hi[pl.ds(j * 16, 16)] = jnp.minimum(lowv + 1, max_len - 1)
        pltpu.async_copy(pe_hbm.at[idx_lo], low_v, sem).wait()
        pltpu.async_copy(pe_hbm.at[idx_hi], high_v, sem).wait()

        def row_body(rr, carry2):
            tvec = jnp.broadcast_to((t0 + rr).astype(jnp.float32), (16,))
            rel = jnp.minimum(tvec * inv_gamma, top)
            w = rel - rel.astype(jnp.int32).astype(jnp.float32)

            def h_body(h, carry3):
                sl = pl.ds(h * 16, 16)
                lo = low_v[rr, sl]
                hi = high_v[rr, sl]
                out_v[rr, sl] = lo + w * (hi - lo)
                return carry3

            return lax.fori_loop(0, 64, h_body, carry2)

        lax.fori_loop(0, _CH, row_body, 0)
        pltpu.sync_copy(out_v, out_hbm.at[pl.ds(t0, _CH)])
        return carry

    lax.fori_loop(0, n_chunks, chunk_body, 0)


def _sc_pe(velocity, pe_base, seq_len):
    max_len, hidden = pe_base.shape
    vel16 = jnp.broadcast_to(velocity, (16,))
    body = functools.partial(_sc_body, max_len=max_len, seq_len=seq_len)
    mesh = plsc.VectorSubcoreMesh(core_axis_name="c", subcore_axis_name="s")
    return pl.kernel(
        body,
        mesh=mesh,
        out_type=jax.ShapeDtypeStruct((seq_len, hidden), jnp.float32),
        scratch_types=[
            pltpu.VMEM((16,), jnp.float32),
            pltpu.VMEM((_CH,), jnp.int32),
            pltpu.VMEM((_CH,), jnp.int32),
            pltpu.VMEM((_CH, hidden), jnp.float32),
            pltpu.VMEM((_CH, hidden), jnp.float32),
            pltpu.VMEM((_CH, hidden), jnp.float32),
            pltpu.SemaphoreType.DMA,
        ],
    )(vel16, pe_base)


def _tc_add(x_ref, pe_ref, out_ref):
    out_ref[...] = x_ref[...] + pe_ref[...][None, :, :]


def kernel(x, velocity, pe_base):
    batch, seq_len, hidden = x.shape
    pe = _sc_pe(velocity, pe_base, seq_len)
    S = 256
    return pl.pallas_call(
        _tc_add,
        grid=(seq_len // S,),
        in_specs=[
            pl.BlockSpec((batch, S, hidden), lambda i: (0, i, 0)),
            pl.BlockSpec((S, hidden), lambda i: (i, 0)),
        ],
        out_specs=pl.BlockSpec((batch, S, hidden), lambda i: (0, i, 0)),
        out_shape=jax.ShapeDtypeStruct((batch, seq_len, hidden), x.dtype),
    )(x, pe)


# speedup vs baseline: 2.2528x; 2.2528x over previous
import functools

import jax
import jax.numpy as jnp
from jax.experimental import pallas as pl
from jax.experimental.pallas import tpu as pltpu


def _slab_start(i, gamma, *, S, SP, max_len):
    p0 = (i * S).astype(jnp.float32)
    rel0 = jnp.clip(p0 / gamma, 0.0, float(max_len - 1))
    a0 = jnp.floor(rel0).astype(jnp.int32)
    a0 = (a0 // 8) * 8
    a0 = jnp.clip(a0, 0, max_len - SP)
    return pl.multiple_of(a0, 8)


def _body(vel_ref, x_ref, pe_hbm, out_ref, rows_ref, sems, *, S, SP, max_len):
    i = pl.program_id(0)
    n = pl.num_programs(0)
    v = jnp.clip(vel_ref[0], 0.0, 0.999)
    gamma = 1.0 / jnp.sqrt(1.0 - v ** 2)
    start = functools.partial(_slab_start, gamma=gamma, S=S, SP=SP,
                              max_len=max_len)

    @pl.when(i == 0)
    def _prologue():
        pltpu.make_async_copy(pe_hbm.at[pl.ds(start(i), SP)],
                              rows_ref.at[0], sems.at[0]).start()

    @pl.when(i + 1 < n)
    def _prefetch():
        pltpu.make_async_copy(pe_hbm.at[pl.ds(start(i + 1), SP)],
                              rows_ref.at[(i + 1) % 2],
                              sems.at[(i + 1) % 2]).start()

    a0 = start(i)
    t = (i * S).astype(jnp.float32) + jax.lax.broadcasted_iota(
        jnp.int32, (S, 1), 0).astype(jnp.float32)
    rel = jnp.clip(t / gamma, 0.0, float(max_len - 1))
    loc = rel - a0.astype(jnp.float32)
    cols = jax.lax.broadcasted_iota(jnp.int32, (S, SP), 1).astype(jnp.float32)
    w = jnp.maximum(1.0 - jnp.abs(loc - cols), 0.0)

    pltpu.make_async_copy(pe_hbm.at[pl.ds(a0, SP)],
                          rows_ref.at[i % 2], sems.at[i % 2]).wait()
    pe = jax.lax.dot_general(
        w, rows_ref[i % 2], (((1,), (0,)), ((), ())),
        preferred_element_type=jnp.float32)
    out_ref[...] = x_ref[...] + pe[None, :, :]


def kernel(x, velocity, pe_base):
    batch, seq_len, hidden = x.shape
    max_len = pe_base.shape[0]
    S = 256
    SP = S + 8
    body = functools.partial(_body, S=S, SP=SP, max_len=max_len)
    return pl.pallas_call(
        body,
        grid=(seq_len // S,),
        in_specs=[
            pl.BlockSpec(memory_space=pltpu.SMEM),
            pl.BlockSpec((batch, S, hidden), lambda i: (0, i, 0)),
            pl.BlockSpec(memory_space=pltpu.MemorySpace.HBM),
        ],
        out_specs=pl.BlockSpec((batch, S, hidden), lambda i: (0, i, 0)),
        out_shape=jax.ShapeDtypeStruct((batch, seq_len, hidden), x.dtype),
        scratch_shapes=[
            pltpu.VMEM((2, SP, hidden), jnp.float32),
            pltpu.SemaphoreType.DMA((2,)),
        ],
    )(velocity, x, pe_base)
